# baseline (speedup 1.0000x reference)
import functools

import jax
import jax.numpy as jnp
from jax import lax
from jax.experimental import pallas as pl
from jax.experimental.pallas import tpu as pltpu
from jax.experimental.pallas import tpu_sc as plsc

N = 10000
E = 100000
D = 128
T = 3
LV = 3
G = 16
O = 64

NC = 2
NS = 16
NW = NC * NS

EPW = E // NW
B = 32
CH = 100
NBUF = 5
EPW_PAD = CH * B
PADW = EPW_PAD - EPW
NPS = 632
ACC_N = NS * NPS
ACC_PAD = ACC_N - N

BLK = 1000
GRID = N // BLK
NP = N + BLK

_f32 = jnp.float32


def _stats_add(st_ref, x, first):
    s1 = jnp.sum(x, axis=0, keepdims=True)
    s2 = jnp.sum(x * x, axis=0, keepdims=True)
    upd = jnp.concatenate([s1, s2, jnp.zeros((6, x.shape[1]), _f32)], axis=0)

    @pl.when(first)
    def _():
        st_ref[...] = jnp.zeros_like(st_ref)

    st_ref[...] += upd


def _bn_of(st_ref, x, g, b):
    mu = st_ref[0:1, :] / N
    var = st_ref[1:2, :] / N - mu * mu
    return (x - mu) * lax.rsqrt(var + 1e-5) * g + b


def _pre_body(nf_ref, w_ref, b_ref, g_ref, bb_ref, cw_ref, cb_ref,
              h_ref, ch0_ref, ch1_ref, ch2_ref, st_s, hpre_s):
    p = pl.program_id(0)
    i = pl.program_id(1)

    @pl.when(p == 0)
    def _():
        h = jnp.tanh(
            jnp.dot(nf_ref[...], w_ref[...], preferred_element_type=_f32)
            + b_ref[...])
        hpre_s[pl.ds(i * BLK, BLK), :] = h
        _stats_add(st_s, h, i == 0)

    @pl.when(p == 1)
    def _():
        h0 = _bn_of(st_s, hpre_s[pl.ds(i * BLK, BLK), :], g_ref[...],
                    bb_ref[...])
        h_ref[...] = h0
        ch = jnp.dot(h0, cw_ref[...], preferred_element_type=_f32) + cb_ref[...]
        ch0_ref[...] = ch[:, 0 * D:1 * D]
        ch1_ref[...] = ch[:, 1 * D:2 * D]
        ch2_ref[...] = ch[:, 2 * D:3 * D]


def _layer_body(part_ref, mw_ref, mb_ref, hg_ref, hb_ref, lw_ref, lb_ref,
                h_ref, g_ref, bb_ref, cw_ref, cb_ref,
                hn_ref, ch0_ref, ch1_ref, ch2_ref,
                st1_s, st2_s, mpre_s, hnp_s):
    p = pl.program_id(0)
    i = pl.program_id(1)

    @pl.when(p == 0)
    def _():
        msg = jnp.concatenate(
            [jnp.tanh(part_ref[0, t] + part_ref[1, t]) for t in range(T)],
            axis=1)
        mp = jnp.dot(msg, mw_ref[...], preferred_element_type=_f32) + mb_ref[...]
        mpre_s[pl.ds(i * BLK, BLK), :] = mp
        _stats_add(st1_s, mp, i == 0)

    @pl.when(p == 1)
    def _():
        merged = _bn_of(st1_s, mpre_s[pl.ds(i * BLK, BLK), :], hg_ref[...],
                        hb_ref[...])
        hn = jnp.tanh(
            jnp.dot(merged, lw_ref[...], preferred_element_type=_f32)
            + lb_ref[...] + h_ref[...])
        hnp_s[pl.ds(i * BLK, BLK), :] = hn
        _stats_add(st2_s, hn, i == 0)

    @pl.when(p == 2)
    def _():
        h3 = _bn_of(st2_s, hnp_s[pl.ds(i * BLK, BLK), :], g_ref[...],
                    bb_ref[...])
        hn_ref[...] = h3
        ch = jnp.dot(h3, cw_ref[...], preferred_element_type=_f32) + cb_ref[...]
        ch0_ref[...] = ch[:, 0 * D:1 * D]
        ch1_ref[...] = ch[:, 1 * D:2 * D]
        ch2_ref[...] = ch[:, 2 * D:3 * D]


def _last_body(part_ref, mw_ref, mb_ref, hg_ref, hb_ref, lw_ref, lb_ref,
               h_ref, g_ref, bb_ref, gidx_ref, rw_ref, rb_ref,
               out_ref, st1_s, st2_s, mpre_s, hnp_s, pooled_s):
    p = pl.program_id(0)
    i = pl.program_id(1)

    @pl.when(p == 0)
    def _():
        msg = jnp.concatenate(
            [jnp.tanh(part_ref[0, t] + part_ref[1, t]) for t in range(T)],
            axis=1)
        mp = jnp.dot(msg, mw_ref[...], preferred_element_type=_f32) + mb_ref[...]
        mpre_s[pl.ds(i * BLK, BLK), :] = mp
        _stats_add(st1_s, mp, i == 0)

    @pl.when(p == 1)
    def _():
        merged = _bn_of(st1_s, mpre_s[pl.ds(i * BLK, BLK), :], hg_ref[...],
                        hb_ref[...])
        hn = jnp.tanh(
            jnp.dot(merged, lw_ref[...], preferred_element_type=_f32)
            + lb_ref[...] + h_ref[...])
        hnp_s[pl.ds(i * BLK, BLK), :] = hn
        _stats_add(st2_s, hn, i == 0)

    @pl.when(p == 2)
    def _():
        h3 = _bn_of(st2_s, hnp_s[pl.ds(i * BLK, BLK), :], g_ref[...],
                    bb_ref[...])
        gcol = gidx_ref[...]
        neg = jnp.full((BLK, D), -jnp.inf, _f32)
        local = jnp.concatenate(
            [jnp.max(jnp.where(gcol == g, h3, neg), axis=0, keepdims=True)
             for g in range(G)], axis=0)
        pooled = jnp.where(i == 0, local, jnp.maximum(pooled_s[...], local))
        pooled_s[...] = pooled

        @pl.when(i == GRID - 1)
        def _():
            out_ref[...] = jnp.tanh(
                jnp.dot(pooled, rw_ref[...], preferred_element_type=_f32)
                + rb_ref[...])


def _const_spec(shape):
    nd = len(shape)
    return pl.BlockSpec(shape, lambda p, i, _n=nd: (0,) * _n)


def _phase_row_spec(phase):
    return pl.BlockSpec(
        (BLK, D), lambda p, i, _ph=phase: (jnp.where(p == _ph, i, GRID), 0))


def _phase_in_spec(phase):
    return pl.BlockSpec(
        (BLK, D), lambda p, i, _ph=phase: (jnp.where(p == _ph, i, 0), 0))


_pre_call = pl.pallas_call(
    _pre_body,
    grid=(2, GRID),
    in_specs=[_phase_in_spec(0), _const_spec((D, D)), _const_spec((1, D)),
              _const_spec((1, D)), _const_spec((1, D)),
              _const_spec((D, T * D)), _const_spec((1, T * D))],
    out_specs=[_phase_row_spec(1)] * 4,
    out_shape=[jax.ShapeDtypeStruct((NP, D), _f32)] * 4,
    scratch_shapes=[pltpu.VMEM((8, D), _f32), pltpu.VMEM((N, D), _f32)],
)

_part_spec = pl.BlockSpec(
    (NC, T, BLK, D), lambda p, i: (0, 0, jnp.where(p == 0, i, 0), 0))

_layer_weight_specs = [
    _const_spec((T * D, D)), _const_spec((1, D)), _const_spec((1, D)),
    _const_spec((1, D)), _const_spec((D, D)), _const_spec((1, D)),
]

_layer_call = pl.pallas_call(
    _layer_body,
    grid=(3, GRID),
    in_specs=[_part_spec] + _layer_weight_specs + [
        _phase_in_spec(1), _const_spec((1, D)), _const_spec((1, D)),
        _const_spec((D, T * D)), _const_spec((1, T * D))],
    out_specs=[_phase_row_spec(2)] * 4,
    out_shape=[jax.ShapeDtypeStruct((NP, D), _f32)] * 4,
    scratch_shapes=[pltpu.VMEM((8, D), _f32), pltpu.VMEM((8, D), _f32),
                    pltpu.VMEM((N, D), _f32), pltpu.VMEM((N, D), _f32)],
)

_last_call = pl.pallas_call(
    _last_body,
    grid=(3, GRID),
    in_specs=[_part_spec] + _layer_weight_specs + [
        _phase_in_spec(1), _const_spec((1, D)), _const_spec((1, D)),
        pl.BlockSpec((BLK, 1), lambda p, i: (jnp.where(p == 2, i, 0), 0)),
        _const_spec((D, O)), _const_spec((1, O))],
    out_specs=_const_spec((G, O)),
    out_shape=jax.ShapeDtypeStruct((G, O), _f32),
    scratch_shapes=[pltpu.VMEM((8, D), _f32), pltpu.VMEM((8, D), _f32),
                    pltpu.VMEM((N, D), _f32), pltpu.VMEM((N, D), _f32),
                    pltpu.VMEM((G, D), _f32)],
)


def _sc_scatter_body(ch0, ch1, ch2, src, dst, zrow, out,
                     idxs_v, idxd_v, rows_v, acc_sh, zsem, wsem, *sems):
    c = lax.axis_index("c")
    s = lax.axis_index("s")
    w = c * NS + s
    chs = (ch0, ch1, ch2)
    stripe = pl.ds(s * NPS, NPS)
    zcopy = pltpu.async_copy(zrow, acc_sh.at[stripe], zsem)
    for t in range(T):
        pltpu.sync_copy(src.at[t, w], idxs_v)
        pltpu.sync_copy(dst.at[t, w], idxd_v)

        ch_t = chs[t]
        for b in range(NBUF):
            pltpu.async_copy(ch_t.at[idxs_v.at[b]], rows_v[b], sems[b])
        zcopy.wait()
        plsc.subcore_barrier()

        def group(i, carry, _ch=ch_t):
            j = NBUF * i
            for b in range(NBUF):
                pltpu.make_async_copy(_ch.at[idxs_v.at[j + b]], rows_v[b],
                                      sems[b]).wait()
                pltpu.sync_copy(rows_v[b], acc_sh.at[idxd_v.at[j + b]],
                                add=True)
                pltpu.async_copy(_ch.at[idxs_v.at[j + NBUF + b]], rows_v[b],
                                 sems[b])
            return carry

        lax.fori_loop(0, CH // NBUF - 1, group, 0)
        for b in range(NBUF):
            j = CH - NBUF + b
            pltpu.make_async_copy(ch_t.at[idxs_v.at[j]], rows_v[b],
                                  sems[b]).wait()
            pltpu.sync_copy(rows_v[b], acc_sh.at[idxd_v.at[j]], add=True)
        plsc.subcore_barrier()
        wcopy = pltpu.async_copy(acc_sh.at[stripe], out.at[c, t, stripe],
                                 wsem)
        wcopy.wait()
        if t < T - 1:
            zcopy = pltpu.async_copy(zrow, acc_sh.at[stripe], zsem)


@functools.cache
def _get_sc_call():
    return pl.kernel(
        _sc_scatter_body,
        out_type=jax.ShapeDtypeStruct((NC, T, ACC_N, D), _f32),
        mesh=plsc.VectorSubcoreMesh(core_axis_name="c", subcore_axis_name="s",
                                    num_cores=NC, num_subcores=NS),
        scratch_types=[
            pltpu.VMEM((CH, B), jnp.int32),
            pltpu.VMEM((CH, B), jnp.int32),
            [pltpu.VMEM((B, D), _f32)] * NBUF,
            pltpu.VMEM_SHARED((ACC_N, D), _f32),
        ] + [pltpu.SemaphoreType.DMA] * (NBUF + 2),
    )


def _prep_edges(edge_index):
    src = edge_index[0].reshape(NW, EPW)
    dst = edge_index[1].reshape(NW, EPW)
    w = jnp.arange(NW, dtype=jnp.int32)[:, None]
    i = jnp.arange(PADW, dtype=jnp.int32)[None, :]
    pad_src = (w * 997 + i * 131) % N
    pad_dst = N + (w * PADW + i) % ACC_PAD
    src = jnp.concatenate([src, pad_src], axis=1).reshape(NW, CH, B)
    dst = jnp.concatenate([dst, pad_dst], axis=1).reshape(NW, CH, B)
    return src, dst


def kernel(node_feat, edge_index_0, edge_index_1, edge_index_2, g_idx,
           w_n2l_W, w_n2l_b, conv_W, conv_b, merge_W, merge_b,
           l2_W, l2_b, msg_bn_g, msg_bn_b, hid_bn_g, hid_bn_b, ro_W, ro_b):
    srcs = []
    dsts = []
    for ei in (edge_index_0, edge_index_1, edge_index_2):
        s_, d_ = _prep_edges(ei)
        srcs.append(s_)
        dsts.append(d_)
    src = jnp.stack(srcs)
    dst = jnp.stack(dsts)
    zrow = jnp.zeros((NPS, D), _f32)

    h, ch0, ch1, ch2 = _pre_call(
        node_feat, w_n2l_W, w_n2l_b.reshape(1, D),
        msg_bn_g[0].reshape(1, D), msg_bn_b[0].reshape(1, D),
        conv_W[0], conv_b[0].reshape(1, T * D))

    for lv in range(LV - 1):
        part = _get_sc_call()(ch0, ch1, ch2, src, dst, zrow)
        h, ch0, ch1, ch2 = _layer_call(
            part, merge_W[lv], merge_b[lv].reshape(1, D),
            hid_bn_g[lv].reshape(1, D), hid_bn_b[lv].reshape(1, D),
            l2_W[lv], l2_b[lv].reshape(1, D), h,
            msg_bn_g[lv + 1].reshape(1, D), msg_bn_b[lv + 1].reshape(1, D),
            conv_W[lv + 1], conv_b[lv + 1].reshape(1, T * D))

    lv = LV - 1
    part = _get_sc_call()(ch0, ch1, ch2, src, dst, zrow)
    return _last_call(
        part, merge_W[lv], merge_b[lv].reshape(1, D),
        hid_bn_g[lv].reshape(1, D), hid_bn_b[lv].reshape(1, D),
        l2_W[lv], l2_b[lv].reshape(1, D), h,
        msg_bn_g[lv + 1].reshape(1, D), msg_bn_b[lv + 1].reshape(1, D),
        g_idx.reshape(N, 1), ro_W, ro_b.reshape(1, O))


# speedup vs baseline: 1.0394x; 1.0394x over previous
import functools

import jax
import jax.numpy as jnp
from jax import lax
from jax.experimental import pallas as pl
from jax.experimental.pallas import tpu as pltpu
from jax.experimental.pallas import tpu_sc as plsc

N = 10000
E = 100000
D = 128
T = 3
LV = 3
G = 16
O = 64

NC = 2
NS = 16
NW = NC * NS

EPW = E // NW
B = 64
CH = 52
NBUF = 4
EPW_PAD = CH * B
PADW = EPW_PAD - EPW
NPS = 632
ACC_N = NS * NPS
ACC_PAD = ACC_N - N

BLK = 1000
GRID = N // BLK
NP = N + BLK

_f32 = jnp.float32


def _stats_add(st_ref, x, first):
    s1 = jnp.sum(x, axis=0, keepdims=True)
    s2 = jnp.sum(x * x, axis=0, keepdims=True)
    upd = jnp.concatenate([s1, s2, jnp.zeros((6, x.shape[1]), _f32)], axis=0)

    @pl.when(first)
    def _():
        st_ref[...] = jnp.zeros_like(st_ref)

    st_ref[...] += upd


def _bn_of(st_ref, x, g, b):
    mu = st_ref[0:1, :] / N
    var = st_ref[1:2, :] / N - mu * mu
    return (x - mu) * lax.rsqrt(var + 1e-5) * g + b


def _pre_body(nf_ref, w_ref, b_ref, g_ref, bb_ref, cw_ref, cb_ref,
              h_ref, ch0_ref, ch1_ref, ch2_ref, st_s, hpre_s):
    p = pl.program_id(0)
    i = pl.program_id(1)

    @pl.when(p == 0)
    def _():
        h = jnp.tanh(
            jnp.dot(nf_ref[...], w_ref[...], preferred_element_type=_f32)
            + b_ref[...])
        hpre_s[pl.ds(i * BLK, BLK), :] = h
        _stats_add(st_s, h, i == 0)

    @pl.when(p == 1)
    def _():
        h0 = _bn_of(st_s, hpre_s[pl.ds(i * BLK, BLK), :], g_ref[...],
                    bb_ref[...])
        h_ref[...] = h0
        ch = jnp.dot(h0, cw_ref[...], preferred_element_type=_f32) + cb_ref[...]
        ch0_ref[...] = ch[:, 0 * D:1 * D]
        ch1_ref[...] = ch[:, 1 * D:2 * D]
        ch2_ref[...] = ch[:, 2 * D:3 * D]


def _layer_body(part_ref, mw_ref, mb_ref, hg_ref, hb_ref, lw_ref, lb_ref,
                h_ref, g_ref, bb_ref, cw_ref, cb_ref,
                hn_ref, ch0_ref, ch1_ref, ch2_ref,
                st1_s, st2_s, mpre_s, hnp_s):
    p = pl.program_id(0)
    i = pl.program_id(1)

    @pl.when(p == 0)
    def _():
        msg = jnp.concatenate(
            [jnp.tanh(part_ref[0, t] + part_ref[1, t]) for t in range(T)],
            axis=1)
        mp = jnp.dot(msg, mw_ref[...], preferred_element_type=_f32) + mb_ref[...]
        mpre_s[pl.ds(i * BLK, BLK), :] = mp
        _stats_add(st1_s, mp, i == 0)

    @pl.when(p == 1)
    def _():
        merged = _bn_of(st1_s, mpre_s[pl.ds(i * BLK, BLK), :], hg_ref[...],
                        hb_ref[...])
        hn = jnp.tanh(
            jnp.dot(merged, lw_ref[...], preferred_element_type=_f32)
            + lb_ref[...] + h_ref[...])
        hnp_s[pl.ds(i * BLK, BLK), :] = hn
        _stats_add(st2_s, hn, i == 0)

    @pl.when(p == 2)
    def _():
        h3 = _bn_of(st2_s, hnp_s[pl.ds(i * BLK, BLK), :], g_ref[...],
                    bb_ref[...])
        hn_ref[...] = h3
        ch = jnp.dot(h3, cw_ref[...], preferred_element_type=_f32) + cb_ref[...]
        ch0_ref[...] = ch[:, 0 * D:1 * D]
        ch1_ref[...] = ch[:, 1 * D:2 * D]
        ch2_ref[...] = ch[:, 2 * D:3 * D]


def _last_body(part_ref, mw_ref, mb_ref, hg_ref, hb_ref, lw_ref, lb_ref,
               h_ref, g_ref, bb_ref, gidx_ref, rw_ref, rb_ref,
               out_ref, st1_s, st2_s, mpre_s, hnp_s, pooled_s):
    p = pl.program_id(0)
    i = pl.program_id(1)

    @pl.when(p == 0)
    def _():
        msg = jnp.concatenate(
            [jnp.tanh(part_ref[0, t] + part_ref[1, t]) for t in range(T)],
            axis=1)
        mp = jnp.dot(msg, mw_ref[...], preferred_element_type=_f32) + mb_ref[...]
        mpre_s[pl.ds(i * BLK, BLK), :] = mp
        _stats_add(st1_s, mp, i == 0)

    @pl.when(p == 1)
    def _():
        merged = _bn_of(st1_s, mpre_s[pl.ds(i * BLK, BLK), :], hg_ref[...],
                        hb_ref[...])
        hn = jnp.tanh(
            jnp.dot(merged, lw_ref[...], preferred_element_type=_f32)
            + lb_ref[...] + h_ref[...])
        hnp_s[pl.ds(i * BLK, BLK), :] = hn
        _stats_add(st2_s, hn, i == 0)

    @pl.when(p == 2)
    def _():
        h3 = _bn_of(st2_s, hnp_s[pl.ds(i * BLK, BLK), :], g_ref[...],
                    bb_ref[...])
        gcol = gidx_ref[...]
        neg = jnp.full((BLK, D), -jnp.inf, _f32)
        local = jnp.concatenate(
            [jnp.max(jnp.where(gcol == g, h3, neg), axis=0, keepdims=True)
             for g in range(G)], axis=0)
        pooled = jnp.where(i == 0, local, jnp.maximum(pooled_s[...], local))
        pooled_s[...] = pooled

        @pl.when(i == GRID - 1)
        def _():
            out_ref[...] = jnp.tanh(
                jnp.dot(pooled, rw_ref[...], preferred_element_type=_f32)
                + rb_ref[...])


def _const_spec(shape):
    nd = len(shape)
    return pl.BlockSpec(shape, lambda p, i, _n=nd: (0,) * _n)


def _phase_row_spec(phase):
    return pl.BlockSpec(
        (BLK, D), lambda p, i, _ph=phase: (jnp.where(p == _ph, i, GRID), 0))


def _phase_in_spec(phase):
    return pl.BlockSpec(
        (BLK, D), lambda p, i, _ph=phase: (jnp.where(p == _ph, i, 0), 0))


_pre_call = pl.pallas_call(
    _pre_body,
    grid=(2, GRID),
    in_specs=[_phase_in_spec(0), _const_spec((D, D)), _const_spec((1, D)),
              _const_spec((1, D)), _const_spec((1, D)),
              _const_spec((D, T * D)), _const_spec((1, T * D))],
    out_specs=[_phase_row_spec(1)] * 4,
    out_shape=[jax.ShapeDtypeStruct((NP, D), _f32)] * 4,
    scratch_shapes=[pltpu.VMEM((8, D), _f32), pltpu.VMEM((N, D), _f32)],
)

_part_spec = pl.BlockSpec(
    (NC, T, BLK, D), lambda p, i: (0, 0, jnp.where(p == 0, i, 0), 0))

_layer_weight_specs = [
    _const_spec((T * D, D)), _const_spec((1, D)), _const_spec((1, D)),
    _const_spec((1, D)), _const_spec((D, D)), _const_spec((1, D)),
]

_layer_call = pl.pallas_call(
    _layer_body,
    grid=(3, GRID),
    in_specs=[_part_spec] + _layer_weight_specs + [
        _phase_in_spec(1), _const_spec((1, D)), _const_spec((1, D)),
        _const_spec((D, T * D)), _const_spec((1, T * D))],
    out_specs=[_phase_row_spec(2)] * 4,
    out_shape=[jax.ShapeDtypeStruct((NP, D), _f32)] * 4,
    scratch_shapes=[pltpu.VMEM((8, D), _f32), pltpu.VMEM((8, D), _f32),
                    pltpu.VMEM((N, D), _f32), pltpu.VMEM((N, D), _f32)],
)

_last_call = pl.pallas_call(
    _last_body,
    grid=(3, GRID),
    in_specs=[_part_spec] + _layer_weight_specs + [
        _phase_in_spec(1), _const_spec((1, D)), _const_spec((1, D)),
        pl.BlockSpec((BLK, 1), lambda p, i: (jnp.where(p == 2, i, 0), 0)),
        _const_spec((D, O)), _const_spec((1, O))],
    out_specs=_const_spec((G, O)),
    out_shape=jax.ShapeDtypeStruct((G, O), _f32),
    scratch_shapes=[pltpu.VMEM((8, D), _f32), pltpu.VMEM((8, D), _f32),
                    pltpu.VMEM((N, D), _f32), pltpu.VMEM((N, D), _f32),
                    pltpu.VMEM((G, D), _f32)],
)


def _sc_scatter_body(ch0, ch1, ch2, src0, src1, src2, dst0, dst1, dst2,
                     zrow, out,
                     idxs_v, idxd_v, rows_v, acc_sh, zsem, wsem, *sems):
    c = lax.axis_index("c")
    s = lax.axis_index("s")
    w = c * NS + s
    chs = (ch0, ch1, ch2)
    srcs = (src0, src1, src2)
    dsts = (dst0, dst1, dst2)
    stripe = pl.ds(s * NPS, NPS)
    zcopy = pltpu.async_copy(zrow, acc_sh.at[stripe], zsem)
    for t in range(T):
        pltpu.sync_copy(srcs[t].at[w], idxs_v)
        pltpu.sync_copy(dsts[t].at[w], idxd_v)

        ch_t = chs[t]
        for b in range(NBUF):
            pltpu.async_copy(ch_t.at[idxs_v.at[b]], rows_v[b], sems[b])
        zcopy.wait()
        plsc.subcore_barrier()

        def group(i, carry, _ch=ch_t):
            j = NBUF * i
            for b in range(NBUF):
                pltpu.make_async_copy(_ch.at[idxs_v.at[j + b]], rows_v[b],
                                      sems[b]).wait()
                pltpu.sync_copy(rows_v[b], acc_sh.at[idxd_v.at[j + b]],
                                add=True)
                pltpu.async_copy(_ch.at[idxs_v.at[j + NBUF + b]], rows_v[b],
                                 sems[b])
            return carry

        lax.fori_loop(0, CH // NBUF - 1, group, 0)
        for b in range(NBUF):
            j = CH - NBUF + b
            pltpu.make_async_copy(ch_t.at[idxs_v.at[j]], rows_v[b],
                                  sems[b]).wait()
            pltpu.sync_copy(rows_v[b], acc_sh.at[idxd_v.at[j]], add=True)
        plsc.subcore_barrier()
        wcopy = pltpu.async_copy(acc_sh.at[stripe], out.at[c, t, stripe],
                                 wsem)
        wcopy.wait()
        if t < T - 1:
            zcopy = pltpu.async_copy(zrow, acc_sh.at[stripe], zsem)


@functools.cache
def _get_sc_call():
    return pl.kernel(
        _sc_scatter_body,
        out_type=jax.ShapeDtypeStruct((NC, T, ACC_N, D), _f32),
        mesh=plsc.VectorSubcoreMesh(core_axis_name="c", subcore_axis_name="s",
                                    num_cores=NC, num_subcores=NS),
        scratch_types=[
            pltpu.VMEM((CH, B), jnp.int32),
            pltpu.VMEM((CH, B), jnp.int32),
            [pltpu.VMEM((B, D), _f32)] * NBUF,
            pltpu.VMEM_SHARED((ACC_N, D), _f32),
        ] + [pltpu.SemaphoreType.DMA] * (NBUF + 2),
    )


def _prep_edges(edge_index):
    src = edge_index[0].reshape(NW, EPW)
    dst = edge_index[1].reshape(NW, EPW)
    w = jnp.arange(NW, dtype=jnp.int32)[:, None]
    i = jnp.arange(PADW, dtype=jnp.int32)[None, :]
    pad_src = (w * 997 + i * 131) % N
    pad_dst = N + (w * PADW + i) % ACC_PAD
    src = jnp.concatenate([src, pad_src], axis=1).reshape(NW, CH, B)
    dst = jnp.concatenate([dst, pad_dst], axis=1).reshape(NW, CH, B)
    return src, dst


def kernel(node_feat, edge_index_0, edge_index_1, edge_index_2, g_idx,
           w_n2l_W, w_n2l_b, conv_W, conv_b, merge_W, merge_b,
           l2_W, l2_b, msg_bn_g, msg_bn_b, hid_bn_g, hid_bn_b, ro_W, ro_b):
    s0, d0 = _prep_edges(edge_index_0)
    s1, d1 = _prep_edges(edge_index_1)
    s2, d2 = _prep_edges(edge_index_2)
    zrow = jnp.zeros((NPS, D), _f32)

    h, ch0, ch1, ch2 = _pre_call(
        node_feat, w_n2l_W, w_n2l_b.reshape(1, D),
        msg_bn_g[0].reshape(1, D), msg_bn_b[0].reshape(1, D),
        conv_W[0], conv_b[0].reshape(1, T * D))

    for lv in range(LV - 1):
        part = _get_sc_call()(ch0, ch1, ch2, s0, s1, s2, d0, d1, d2, zrow)
        h, ch0, ch1, ch2 = _layer_call(
            part, merge_W[lv], merge_b[lv].reshape(1, D),
            hid_bn_g[lv].reshape(1, D), hid_bn_b[lv].reshape(1, D),
            l2_W[lv], l2_b[lv].reshape(1, D), h,
            msg_bn_g[lv + 1].reshape(1, D), msg_bn_b[lv + 1].reshape(1, D),
            conv_W[lv + 1], conv_b[lv + 1].reshape(1, T * D))

    lv = LV - 1
    part = _get_sc_call()(ch0, ch1, ch2, s0, s1, s2, d0, d1, d2, zrow)
    return _last_call(
        part, merge_W[lv], merge_b[lv].reshape(1, D),
        hid_bn_g[lv].reshape(1, D), hid_bn_b[lv].reshape(1, D),
        l2_W[lv], l2_b[lv].reshape(1, D), h,
        msg_bn_g[lv + 1].reshape(1, D), msg_bn_b[lv + 1].reshape(1, D),
        g_idx.reshape(N, 1), ro_W, ro_b.reshape(1, O))


# speedup vs baseline: 1.1293x; 1.0865x over previous
import functools

import jax
import jax.numpy as jnp
from jax import lax
from jax.experimental import pallas as pl
from jax.experimental.pallas import tpu as pltpu
from jax.experimental.pallas import tpu_sc as plsc

N = 10000
E = 100000
D = 128
T = 3
LV = 3
G = 16
O = 64

NC = 2
NS = 16
NW = NC * NS

EPW = E // NW
B = 64
CH = 52
NBUF = 4
EPW_PAD = CH * B
PADW = EPW_PAD - EPW
NPS = 632
ACC_N = NS * NPS
ACC_PAD = ACC_N - N

BLK = 1000
GRID = N // BLK
NP = N + BLK

_f32 = jnp.float32


def _stats_add(st_ref, x, first):
    s1 = jnp.sum(x, axis=0, keepdims=True)
    s2 = jnp.sum(x * x, axis=0, keepdims=True)
    upd = jnp.concatenate([s1, s2, jnp.zeros((6, x.shape[1]), _f32)], axis=0)

    @pl.when(first)
    def _():
        st_ref[...] = jnp.zeros_like(st_ref)

    st_ref[...] += upd


def _bn_of(st_ref, x, g, b):
    mu = st_ref[0:1, :] / N
    var = st_ref[1:2, :] / N - mu * mu
    return (x - mu) * lax.rsqrt(var + 1e-5) * g + b


def _pre_body(nf_ref, w_ref, b_ref, g_ref, bb_ref, cw_ref, cb_ref,
              h_ref, ch0_ref, ch1_ref, ch2_ref, st_s, hpre_s):
    p = pl.program_id(0)
    i = pl.program_id(1)

    @pl.when(p == 0)
    def _():
        h = jnp.tanh(
            jnp.dot(nf_ref[...], w_ref[...], preferred_element_type=_f32)
            + b_ref[...])
        hpre_s[pl.ds(i * BLK, BLK), :] = h
        _stats_add(st_s, h, i == 0)

    @pl.when(p == 1)
    def _():
        h0 = _bn_of(st_s, hpre_s[pl.ds(i * BLK, BLK), :], g_ref[...],
                    bb_ref[...])
        h_ref[...] = h0
        ch = jnp.dot(h0, cw_ref[...], preferred_element_type=_f32) + cb_ref[...]
        ch0_ref[...] = ch[:, 0 * D:1 * D]
        ch1_ref[...] = ch[:, 1 * D:2 * D]
        ch2_ref[...] = ch[:, 2 * D:3 * D]


def _layer_body(part_ref, mw_ref, mb_ref, hg_ref, hb_ref, lw_ref, lb_ref,
                h_ref, g_ref, bb_ref, cw_ref, cb_ref,
                hn_ref, ch0_ref, ch1_ref, ch2_ref,
                st1_s, st2_s, mpre_s, hnp_s):
    p = pl.program_id(0)
    i = pl.program_id(1)

    @pl.when(p == 0)
    def _():
        msgs = []
        for t in range(T):
            m = part_ref[0, t] + part_ref[1, t]
            if t > 0:
                m = m - (part_ref[0, t - 1] + part_ref[1, t - 1])
            msgs.append(jnp.tanh(m))
        msg = jnp.concatenate(msgs, axis=1)
        mp = jnp.dot(msg, mw_ref[...], preferred_element_type=_f32) + mb_ref[...]
        mpre_s[pl.ds(i * BLK, BLK), :] = mp
        _stats_add(st1_s, mp, i == 0)

    @pl.when(p == 1)
    def _():
        merged = _bn_of(st1_s, mpre_s[pl.ds(i * BLK, BLK), :], hg_ref[...],
                        hb_ref[...])
        hn = jnp.tanh(
            jnp.dot(merged, lw_ref[...], preferred_element_type=_f32)
            + lb_ref[...] + h_ref[...])
        hnp_s[pl.ds(i * BLK, BLK), :] = hn
        _stats_add(st2_s, hn, i == 0)

    @pl.when(p == 2)
    def _():
        h3 = _bn_of(st2_s, hnp_s[pl.ds(i * BLK, BLK), :], g_ref[...],
                    bb_ref[...])
        hn_ref[...] = h3
        ch = jnp.dot(h3, cw_ref[...], preferred_element_type=_f32) + cb_ref[...]
        ch0_ref[...] = ch[:, 0 * D:1 * D]
        ch1_ref[...] = ch[:, 1 * D:2 * D]
        ch2_ref[...] = ch[:, 2 * D:3 * D]


def _last_body(part_ref, mw_ref, mb_ref, hg_ref, hb_ref, lw_ref, lb_ref,
               h_ref, g_ref, bb_ref, gidx_ref, rw_ref, rb_ref,
               out_ref, st1_s, st2_s, mpre_s, hnp_s, pooled_s):
    p = pl.program_id(0)
    i = pl.program_id(1)

    @pl.when(p == 0)
    def _():
        msgs = []
        for t in range(T):
            m = part_ref[0, t] + part_ref[1, t]
            if t > 0:
                m = m - (part_ref[0, t - 1] + part_ref[1, t - 1])
            msgs.append(jnp.tanh(m))
        msg = jnp.concatenate(msgs, axis=1)
        mp = jnp.dot(msg, mw_ref[...], preferred_element_type=_f32) + mb_ref[...]
        mpre_s[pl.ds(i * BLK, BLK), :] = mp
        _stats_add(st1_s, mp, i == 0)

    @pl.when(p == 1)
    def _():
        merged = _bn_of(st1_s, mpre_s[pl.ds(i * BLK, BLK), :], hg_ref[...],
                        hb_ref[...])
        hn = jnp.tanh(
            jnp.dot(merged, lw_ref[...], preferred_element_type=_f32)
            + lb_ref[...] + h_ref[...])
        hnp_s[pl.ds(i * BLK, BLK), :] = hn
        _stats_add(st2_s, hn, i == 0)

    @pl.when(p == 2)
    def _():
        h3 = _bn_of(st2_s, hnp_s[pl.ds(i * BLK, BLK), :], g_ref[...],
                    bb_ref[...])
        gcol = gidx_ref[...]
        neg = jnp.full((BLK, D), -jnp.inf, _f32)
        local = jnp.concatenate(
            [jnp.max(jnp.where(gcol == g, h3, neg), axis=0, keepdims=True)
             for g in range(G)], axis=0)
        pooled = jnp.where(i == 0, local, jnp.maximum(pooled_s[...], local))
        pooled_s[...] = pooled

        @pl.when(i == GRID - 1)
        def _():
            out_ref[...] = jnp.tanh(
                jnp.dot(pooled, rw_ref[...], preferred_element_type=_f32)
                + rb_ref[...])


def _const_spec(shape):
    nd = len(shape)
    return pl.BlockSpec(shape, lambda p, i, _n=nd: (0,) * _n)


def _phase_row_spec(phase):
    return pl.BlockSpec(
        (BLK, D), lambda p, i, _ph=phase: (jnp.where(p == _ph, i, GRID), 0))


def _phase_in_spec(phase):
    return pl.BlockSpec(
        (BLK, D), lambda p, i, _ph=phase: (jnp.where(p == _ph, i, 0), 0))


_pre_call = pl.pallas_call(
    _pre_body,
    grid=(2, GRID),
    in_specs=[_phase_in_spec(0), _const_spec((D, D)), _const_spec((1, D)),
              _const_spec((1, D)), _const_spec((1, D)),
              _const_spec((D, T * D)), _const_spec((1, T * D))],
    out_specs=[_phase_row_spec(1)] * 4,
    out_shape=[jax.ShapeDtypeStruct((NP, D), _f32)] * 4,
    scratch_shapes=[pltpu.VMEM((8, D), _f32), pltpu.VMEM((N, D), _f32)],
)

_part_spec = pl.BlockSpec(
    (NC, T, BLK, D), lambda p, i: (0, 0, jnp.where(p == 0, i, 0), 0))

_layer_weight_specs = [
    _const_spec((T * D, D)), _const_spec((1, D)), _const_spec((1, D)),
    _const_spec((1, D)), _const_spec((D, D)), _const_spec((1, D)),
]

_layer_call = pl.pallas_call(
    _layer_body,
    grid=(3, GRID),
    in_specs=[_part_spec] + _layer_weight_specs + [
        _phase_in_spec(1), _const_spec((1, D)), _const_spec((1, D)),
        _const_spec((D, T * D)), _const_spec((1, T * D))],
    out_specs=[_phase_row_spec(2)] * 4,
    out_shape=[jax.ShapeDtypeStruct((NP, D), _f32)] * 4,
    scratch_shapes=[pltpu.VMEM((8, D), _f32), pltpu.VMEM((8, D), _f32),
                    pltpu.VMEM((N, D), _f32), pltpu.VMEM((N, D), _f32)],
)

_last_call = pl.pallas_call(
    _last_body,
    grid=(3, GRID),
    in_specs=[_part_spec] + _layer_weight_specs + [
        _phase_in_spec(1), _const_spec((1, D)), _const_spec((1, D)),
        pl.BlockSpec((BLK, 1), lambda p, i: (jnp.where(p == 2, i, 0), 0)),
        _const_spec((D, O)), _const_spec((1, O))],
    out_specs=_const_spec((G, O)),
    out_shape=jax.ShapeDtypeStruct((G, O), _f32),
    scratch_shapes=[pltpu.VMEM((8, D), _f32), pltpu.VMEM((8, D), _f32),
                    pltpu.VMEM((N, D), _f32), pltpu.VMEM((N, D), _f32),
                    pltpu.VMEM((G, D), _f32)],
)


def _sc_scatter_body(ch0, ch1, ch2, src0, src1, src2, dst0, dst1, dst2,
                     zrow, out,
                     idxs_v, idxd_v, rows_v, acc_sh, zsem, wsem, *sems):
    c = lax.axis_index("c")
    s = lax.axis_index("s")
    w = c * NS + s
    chs = (ch0, ch1, ch2)
    srcs = (src0, src1, src2)
    dsts = (dst0, dst1, dst2)
    stripe = pl.ds(s * NPS, NPS)
    zcopy = pltpu.async_copy(zrow, acc_sh.at[stripe], zsem)
    wcopy = None
    for t in range(T):
        pltpu.sync_copy(srcs[t].at[w], idxs_v)
        pltpu.sync_copy(dsts[t].at[w], idxd_v)

        ch_t = chs[t]
        for b in range(NBUF):
            pltpu.async_copy(ch_t.at[idxs_v.at[b]], rows_v[b], sems[b])
        if t == 0:
            zcopy.wait()
        else:
            wcopy.wait()
        plsc.subcore_barrier()

        def group(i, carry, _ch=ch_t):
            j = NBUF * i
            for b in range(NBUF):
                pltpu.make_async_copy(_ch.at[idxs_v.at[j + b]], rows_v[b],
                                      sems[b]).wait()
                pltpu.sync_copy(rows_v[b], acc_sh.at[idxd_v.at[j + b]],
                                add=True)
                pltpu.async_copy(_ch.at[idxs_v.at[j + NBUF + b]], rows_v[b],
                                 sems[b])
            return carry

        lax.fori_loop(0, CH // NBUF - 1, group, 0)
        for b in range(NBUF):
            j = CH - NBUF + b
            pltpu.make_async_copy(ch_t.at[idxs_v.at[j]], rows_v[b],
                                  sems[b]).wait()
            pltpu.sync_copy(rows_v[b], acc_sh.at[idxd_v.at[j]], add=True)
        plsc.subcore_barrier()
        wcopy = pltpu.async_copy(acc_sh.at[stripe], out.at[c, t, stripe],
                                 wsem)
    wcopy.wait()


@functools.cache
def _get_sc_call():
    return pl.kernel(
        _sc_scatter_body,
        out_type=jax.ShapeDtypeStruct((NC, T, ACC_N, D), _f32),
        mesh=plsc.VectorSubcoreMesh(core_axis_name="c", subcore_axis_name="s",
                                    num_cores=NC, num_subcores=NS),
        scratch_types=[
            pltpu.VMEM((CH, B), jnp.int32),
            pltpu.VMEM((CH, B), jnp.int32),
            [pltpu.VMEM((B, D), _f32)] * NBUF,
            pltpu.VMEM_SHARED((ACC_N, D), _f32),
        ] + [pltpu.SemaphoreType.DMA] * (NBUF + 2),
    )


def _prep_edges(edge_index):
    src = edge_index[0].reshape(NW, EPW)
    dst = edge_index[1].reshape(NW, EPW)
    w = jnp.arange(NW, dtype=jnp.int32)[:, None]
    i = jnp.arange(PADW, dtype=jnp.int32)[None, :]
    pad_src = (w * 997 + i * 131) % N
    pad_dst = N + (w * PADW + i) % ACC_PAD
    src = jnp.concatenate([src, pad_src], axis=1).reshape(NW, CH, B)
    dst = jnp.concatenate([dst, pad_dst], axis=1).reshape(NW, CH, B)
    return src, dst


def kernel(node_feat, edge_index_0, edge_index_1, edge_index_2, g_idx,
           w_n2l_W, w_n2l_b, conv_W, conv_b, merge_W, merge_b,
           l2_W, l2_b, msg_bn_g, msg_bn_b, hid_bn_g, hid_bn_b, ro_W, ro_b):
    s0, d0 = _prep_edges(edge_index_0)
    s1, d1 = _prep_edges(edge_index_1)
    s2, d2 = _prep_edges(edge_index_2)
    zrow = jnp.zeros((NPS, D), _f32)

    h, ch0, ch1, ch2 = _pre_call(
        node_feat, w_n2l_W, w_n2l_b.reshape(1, D),
        msg_bn_g[0].reshape(1, D), msg_bn_b[0].reshape(1, D),
        conv_W[0], conv_b[0].reshape(1, T * D))

    for lv in range(LV - 1):
        part = _get_sc_call()(ch0, ch1, ch2, s0, s1, s2, d0, d1, d2, zrow)
        h, ch0, ch1, ch2 = _layer_call(
            part, merge_W[lv], merge_b[lv].reshape(1, D),
            hid_bn_g[lv].reshape(1, D), hid_bn_b[lv].reshape(1, D),
            l2_W[lv], l2_b[lv].reshape(1, D), h,
            msg_bn_g[lv + 1].reshape(1, D), msg_bn_b[lv + 1].reshape(1, D),
            conv_W[lv + 1], conv_b[lv + 1].reshape(1, T * D))

    lv = LV - 1
    part = _get_sc_call()(ch0, ch1, ch2, s0, s1, s2, d0, d1, d2, zrow)
    return _last_call(
        part, merge_W[lv], merge_b[lv].reshape(1, D),
        hid_bn_g[lv].reshape(1, D), hid_bn_b[lv].reshape(1, D),
        l2_W[lv], l2_b[lv].reshape(1, D), h,
        msg_bn_g[lv + 1].reshape(1, D), msg_bn_b[lv + 1].reshape(1, D),
        g_idx.reshape(N, 1), ro_W, ro_b.reshape(1, O))
